# baseline (device time: 31512 ns/iter reference)
import jax
import jax.numpy as jnp
from jax import lax
from jax.experimental import pallas as pl
from jax.experimental.pallas import tpu as pltpu

N_DEV = 32
C_PER = 4
LOG_M = 8
LOG_TOT = 13


def kernel(x):
    m, n = x.shape
    assert m == 256 and n == 128
    wn, wm = n, m

    def body(x_ref, o_ref, xt_ref, w_ref, ot_ref, sf_send, sf_recv, sb_send, sb_recv):
        pos = lax.axis_index("i")

        def drain(send_sem, recv_sem, count, which):
            dummy = pltpu.make_async_remote_copy(
                src_ref=xt_ref.at[pl.ds(0, C_PER)],
                dst_ref=xt_ref.at[pl.ds(0, C_PER)],
                send_sem=send_sem,
                recv_sem=recv_sem,
                device_id=(pos,),
                device_id_type=pl.DeviceIdType.MESH,
            )
            for _ in range(count):
                if which == "recv":
                    dummy.wait_recv()
                else:
                    dummy.wait_send()

        def cmp_exchange(w, j, asc, up, down, lower):
            partner = jnp.where(lower, up, down)
            return jnp.where(
                asc == lower,
                jnp.minimum(w, partner),
                jnp.maximum(w, partner),
            )

        barrier_sem = pltpu.get_barrier_semaphore()
        for d in range(1, N_DEV):
            pl.semaphore_signal(
                barrier_sem,
                inc=1,
                device_id=(pos ^ d,),
                device_id_type=pl.DeviceIdType.MESH,
            )

        xt_ref[:, :] = x_ref[:, :].T

        pl.semaphore_wait(barrier_sem, N_DEV - 1)

        for d in range(N_DEV):
            q = pos ^ d
            copy = pltpu.make_async_remote_copy(
                src_ref=xt_ref.at[pl.ds(C_PER * q, C_PER)],
                dst_ref=w_ref.at[pl.ds(C_PER * pos, C_PER)],
                send_sem=sf_send,
                recv_sem=sf_recv,
                device_id=(q,),
                device_id_type=pl.DeviceIdType.MESH,
            )
            copy.start()

        drain(sf_send, sf_recv, N_DEV, "recv")

        row = lax.broadcasted_iota(jnp.int32, (wn, wm), 0)
        lane = lax.broadcasted_iota(jnp.int32, (wn, wm), 1)
        fidx = (row // C_PER) * wm + lane

        w = w_ref[:, :]
        for lk in range(1, LOG_TOT + 1):
            k = 1 << lk
            asc = (fidx & k) == 0
            for lj in range(lk - 1, -1, -1):
                j = 1 << lj
                if j >= wm:
                    dd = C_PER * (j // wm)
                    up = pltpu.roll(w, wn - dd, 0)
                    down = pltpu.roll(w, dd, 0)
                else:
                    up = pltpu.roll(w, wm - j, 1)
                    down = pltpu.roll(w, j, 1)
                w = cmp_exchange(w, j, asc, up, down, (fidx & j) == 0)
        w_ref[:, :] = w

        for d in range(N_DEV):
            p = pos ^ d
            copy = pltpu.make_async_remote_copy(
                src_ref=w_ref.at[pl.ds(C_PER * p, C_PER)],
                dst_ref=ot_ref.at[pl.ds(C_PER * pos, C_PER)],
                send_sem=sb_send,
                recv_sem=sb_recv,
                device_id=(p,),
                device_id_type=pl.DeviceIdType.MESH,
            )
            copy.start()

        drain(sb_send, sb_recv, N_DEV, "recv")
        o_ref[:, :] = ot_ref[:, :].T

        drain(sf_send, sf_recv, N_DEV, "send")
        drain(sb_send, sb_recv, N_DEV, "send")

    return pl.pallas_call(
        body,
        out_shape=jax.ShapeDtypeStruct((m, n), x.dtype),
        in_specs=[pl.BlockSpec(memory_space=pltpu.VMEM)],
        out_specs=pl.BlockSpec(memory_space=pltpu.VMEM),
        scratch_shapes=[
            pltpu.VMEM((wn, wm), x.dtype),
            pltpu.VMEM((wn, wm), x.dtype),
            pltpu.VMEM((wn, wm), x.dtype),
            pltpu.SemaphoreType.DMA,
            pltpu.SemaphoreType.DMA,
            pltpu.SemaphoreType.DMA,
            pltpu.SemaphoreType.DMA,
        ],
        compiler_params=pltpu.CompilerParams(collective_id=0),
    )(x)


# device time: 31262 ns/iter; 1.0080x vs baseline; 1.0080x over previous
import jax
import jax.numpy as jnp
from jax import lax
from jax.experimental import pallas as pl
from jax.experimental.pallas import tpu as pltpu

N_DEV = 32
C_PER = 4
LOG_M = 8
LOG_TOT = 13


def kernel(x):
    m, n = x.shape
    assert m == 256 and n == 128
    wn, wm = n, m

    def body(x_ref, o_ref, xt_ref, w_ref, ot_ref, sf_send, sf_recv, sb_send, sb_recv):
        pos = lax.axis_index("i")

        def drain(send_sem, recv_sem, count, which):
            dummy = pltpu.make_async_remote_copy(
                src_ref=xt_ref.at[pl.ds(0, C_PER)],
                dst_ref=xt_ref.at[pl.ds(0, C_PER)],
                send_sem=send_sem,
                recv_sem=recv_sem,
                device_id=(pos,),
                device_id_type=pl.DeviceIdType.MESH,
            )
            for _ in range(count):
                if which == "recv":
                    dummy.wait_recv()
                else:
                    dummy.wait_send()

        def cmp_exchange(w, j, asc, up, down, lower):
            partner = jnp.where(lower, up, down)
            return jnp.where(
                asc == lower,
                jnp.minimum(w, partner),
                jnp.maximum(w, partner),
            )

        barrier_sem = pltpu.get_barrier_semaphore()
        for d in range(1, N_DEV):
            pl.semaphore_signal(
                barrier_sem,
                inc=1,
                device_id=(pos ^ d,),
                device_id_type=pl.DeviceIdType.MESH,
            )

        xt_ref[:, :] = x_ref[:, :].T

        pl.semaphore_wait(barrier_sem, N_DEV - 1)

        for d in range(N_DEV):
            q = pos ^ d
            copy = pltpu.make_async_remote_copy(
                src_ref=xt_ref.at[pl.ds(C_PER * q, C_PER)],
                dst_ref=w_ref.at[pl.ds(C_PER * pos, C_PER)],
                send_sem=sf_send,
                recv_sem=sf_recv,
                device_id=(q,),
                device_id_type=pl.DeviceIdType.MESH,
            )
            copy.start()

        drain(sf_send, sf_recv, N_DEV, "recv")

        row = lax.broadcasted_iota(jnp.int32, (wn, wm), 0)
        lane = lax.broadcasted_iota(jnp.int32, (wn, wm), 1)
        fidx = (row // C_PER) * wm + lane

        w = w_ref[:, :]
        for lk in range(1, LOG_TOT + 1):
            k = 1 << lk
            asc = (fidx & k) == 0
            for lj in range(lk - 1, -1, -1):
                j = 1 << lj
                if j >= wm:
                    dd = C_PER * (j // wm)
                    up = jnp.roll(w, -dd, axis=0)
                    down = jnp.roll(w, dd, axis=0)
                else:
                    up = jnp.roll(w, -j, axis=1)
                    down = jnp.roll(w, j, axis=1)
                w = cmp_exchange(w, j, asc, up, down, (fidx & j) == 0)
        w_ref[:, :] = w

        for d in range(N_DEV):
            p = pos ^ d
            copy = pltpu.make_async_remote_copy(
                src_ref=w_ref.at[pl.ds(C_PER * p, C_PER)],
                dst_ref=ot_ref.at[pl.ds(C_PER * pos, C_PER)],
                send_sem=sb_send,
                recv_sem=sb_recv,
                device_id=(p,),
                device_id_type=pl.DeviceIdType.MESH,
            )
            copy.start()

        drain(sb_send, sb_recv, N_DEV, "recv")
        o_ref[:, :] = ot_ref[:, :].T

        drain(sf_send, sf_recv, N_DEV, "send")
        drain(sb_send, sb_recv, N_DEV, "send")

    return pl.pallas_call(
        body,
        out_shape=jax.ShapeDtypeStruct((m, n), x.dtype),
        in_specs=[pl.BlockSpec(memory_space=pltpu.VMEM)],
        out_specs=pl.BlockSpec(memory_space=pltpu.VMEM),
        scratch_shapes=[
            pltpu.VMEM((wn, wm), x.dtype),
            pltpu.VMEM((wn, wm), x.dtype),
            pltpu.VMEM((wn, wm), x.dtype),
            pltpu.SemaphoreType.DMA,
            pltpu.SemaphoreType.DMA,
            pltpu.SemaphoreType.DMA,
            pltpu.SemaphoreType.DMA,
        ],
        compiler_params=pltpu.CompilerParams(collective_id=0),
    )(x)


# device time: 22313 ns/iter; 1.4123x vs baseline; 1.4011x over previous
import jax
import jax.numpy as jnp
from jax import lax
from jax.experimental import pallas as pl
from jax.experimental.pallas import tpu as pltpu

N_DEV = 32
C_PER = 4
LOG_M = 8
LOG_TOT = 13


def kernel(x):
    m, n = x.shape
    assert m == 256 and n == 128
    wn, wm = n, m

    def body(x_ref, o_ref, xt_ref, w_ref, ot_ref, sf_send, sf_recv, sb_send, sb_recv):
        pos = lax.axis_index("i")

        def drain(send_sem, recv_sem, count, which):
            dummy = pltpu.make_async_remote_copy(
                src_ref=xt_ref.at[pl.ds(0, C_PER)],
                dst_ref=xt_ref.at[pl.ds(0, C_PER)],
                send_sem=send_sem,
                recv_sem=recv_sem,
                device_id=(pos,),
                device_id_type=pl.DeviceIdType.MESH,
            )
            for _ in range(count):
                if which == "recv":
                    dummy.wait_recv()
                else:
                    dummy.wait_send()

        def cmp_exchange(w, j, asc, up, down, lower):
            if asc is True:
                return jnp.where(
                    lower, jnp.minimum(w, up), jnp.maximum(w, down)
                )
            partner = jnp.where(lower, up, down)
            return jnp.where(
                asc == lower,
                jnp.minimum(w, partner),
                jnp.maximum(w, partner),
            )

        barrier_sem = pltpu.get_barrier_semaphore()
        for d in range(1, N_DEV):
            pl.semaphore_signal(
                barrier_sem,
                inc=1,
                device_id=(pos ^ d,),
                device_id_type=pl.DeviceIdType.MESH,
            )

        xv = x_ref[:, :].T
        lane_s = lax.broadcasted_iota(jnp.int32, (wn, wm), 1)
        for lk in range(1, 9):
            k = 1 << lk
            asc = ((lane_s & k) == 0) if k < wm else ((pos & 1) == 0)
            for lj in range(lk - 1, -1, -1):
                j = 1 << lj
                up = jnp.roll(xv, -j, axis=1)
                down = jnp.roll(xv, j, axis=1)
                xv = cmp_exchange(xv, j, asc, up, down, (lane_s & j) == 0)
        xt_ref[:, :] = xv

        pl.semaphore_wait(barrier_sem, N_DEV - 1)

        for d in range(N_DEV):
            q = pos ^ d
            copy = pltpu.make_async_remote_copy(
                src_ref=xt_ref.at[pl.ds(C_PER * q, C_PER)],
                dst_ref=w_ref.at[pl.ds(C_PER * pos, C_PER)],
                send_sem=sf_send,
                recv_sem=sf_recv,
                device_id=(q,),
                device_id_type=pl.DeviceIdType.MESH,
            )
            copy.start()

        drain(sf_send, sf_recv, N_DEV, "recv")

        row = lax.broadcasted_iota(jnp.int32, (wn, wm), 0)
        lane = lax.broadcasted_iota(jnp.int32, (wn, wm), 1)
        fidx = (row // C_PER) * wm + lane

        w = w_ref[:, :]
        for lk in range(9, LOG_TOT + 1):
            k = 1 << lk
            asc = True if lk == LOG_TOT else (fidx & k) == 0
            for lj in range(lk - 1, -1, -1):
                j = 1 << lj
                if j >= wm:
                    dd = C_PER * (j // wm)
                    up = jnp.roll(w, -dd, axis=0)
                    down = jnp.roll(w, dd, axis=0)
                else:
                    up = jnp.roll(w, -j, axis=1)
                    down = jnp.roll(w, j, axis=1)
                w = cmp_exchange(w, j, asc, up, down, (fidx & j) == 0)
        w_ref[:, :] = w

        for d in range(N_DEV):
            p = pos ^ d
            copy = pltpu.make_async_remote_copy(
                src_ref=w_ref.at[pl.ds(C_PER * p, C_PER)],
                dst_ref=ot_ref.at[pl.ds(C_PER * pos, C_PER)],
                send_sem=sb_send,
                recv_sem=sb_recv,
                device_id=(p,),
                device_id_type=pl.DeviceIdType.MESH,
            )
            copy.start()

        drain(sb_send, sb_recv, N_DEV, "recv")
        o_ref[:, :] = ot_ref[:, :].T

        drain(sf_send, sf_recv, N_DEV, "send")
        drain(sb_send, sb_recv, N_DEV, "send")

    return pl.pallas_call(
        body,
        out_shape=jax.ShapeDtypeStruct((m, n), x.dtype),
        in_specs=[pl.BlockSpec(memory_space=pltpu.VMEM)],
        out_specs=pl.BlockSpec(memory_space=pltpu.VMEM),
        scratch_shapes=[
            pltpu.VMEM((wn, wm), x.dtype),
            pltpu.VMEM((wn, wm), x.dtype),
            pltpu.VMEM((wn, wm), x.dtype),
            pltpu.SemaphoreType.DMA,
            pltpu.SemaphoreType.DMA,
            pltpu.SemaphoreType.DMA,
            pltpu.SemaphoreType.DMA,
        ],
        compiler_params=pltpu.CompilerParams(collective_id=0),
    )(x)
